# baseline (speedup 1.0000x reference)
import jax
import jax.numpy as jnp
from jax.experimental import pallas as pl

NUM_CLASSES_K = 1000
BATCH_K = 4096
COLS_K = 20


CBLK_K = 200


def _onehot_body(xt_ref, o_ref):
    c = pl.program_id(1)
    xv = xt_ref[...]
    classes = c * CBLK_K + jax.lax.broadcasted_iota(
        jnp.int32, (1, CBLK_K, BATCH_K), 1
    )
    o_ref[...] = (xv == classes).astype(jnp.float32)


def kernel(x):
    xt = x.astype(jnp.int32).T.reshape(COLS_K, 1, BATCH_K)
    out = pl.pallas_call(
        _onehot_body,
        grid=(COLS_K, NUM_CLASSES_K // CBLK_K),
        in_specs=[pl.BlockSpec((1, 1, BATCH_K), lambda t, c: (t, 0, 0))],
        out_specs=pl.BlockSpec((1, CBLK_K, BATCH_K), lambda t, c: (t, c, 0)),
        out_shape=jax.ShapeDtypeStruct((COLS_K, NUM_CLASSES_K, BATCH_K), jnp.float32),
    )(xt)
    return out.transpose(2, 0, 1)


# speedup vs baseline: 1.0223x; 1.0223x over previous
import jax
import jax.numpy as jnp
from jax.experimental import pallas as pl

NUM_CLASSES_K = 1000
BATCH_K = 4096
COLS_K = 20


def _onehot_body(xt_ref, o_ref):
    t = pl.program_id(0)
    xv = xt_ref[pl.ds(t, 1), :]
    classes = jax.lax.broadcasted_iota(jnp.int32, (NUM_CLASSES_K, BATCH_K), 0)
    o_ref[...] = (xv == classes).astype(jnp.float32)[None]


def kernel(x):
    xt = x.astype(jnp.int32).T
    out = pl.pallas_call(
        _onehot_body,
        grid=(COLS_K,),
        in_specs=[pl.BlockSpec((COLS_K, BATCH_K), lambda t: (0, 0))],
        out_specs=pl.BlockSpec((1, NUM_CLASSES_K, BATCH_K), lambda t: (t, 0, 0)),
        out_shape=jax.ShapeDtypeStruct((COLS_K, NUM_CLASSES_K, BATCH_K), jnp.float32),
    )(xt)
    return out.transpose(2, 0, 1)


# speedup vs baseline: 1.0282x; 1.0058x over previous
import jax
import jax.numpy as jnp
from jax.experimental import pallas as pl

NUM_CLASSES_K = 1000
BATCH_K = 4096
COLS_K = 20


def _onehot_body(xt_ref, o_ref):
    t = pl.program_id(0)
    xv = xt_ref[pl.ds(t, 1), :]
    classes = jax.lax.broadcasted_iota(jnp.int32, (NUM_CLASSES_K, BATCH_K), 0)
    o_ref[...] = jnp.where(
        xv == classes, jnp.float32(1.0), jnp.float32(0.0)
    )[None]


def kernel(x):
    xt = x.astype(jnp.int32).T
    out = pl.pallas_call(
        _onehot_body,
        grid=(COLS_K,),
        in_specs=[pl.BlockSpec((COLS_K, BATCH_K), lambda t: (0, 0))],
        out_specs=pl.BlockSpec((1, NUM_CLASSES_K, BATCH_K), lambda t: (t, 0, 0)),
        out_shape=jax.ShapeDtypeStruct((COLS_K, NUM_CLASSES_K, BATCH_K), jnp.float32),
    )(xt)
    return out.transpose(2, 0, 1)
